# baseline (speedup 1.0000x reference)
import functools

import jax
import jax.numpy as jnp
from jax import lax
from jax.experimental import pallas as pl
from jax.experimental.pallas import tpu as pltpu
from jax.experimental.pallas import tpu_sc as plsc

NC = 2
NS = 16
LANES = 16
CHUNK = 128
GRP = 8


def _ceil_to(x, m):
    return (x + m - 1) // m * m


@functools.lru_cache(maxsize=None)
def _make_sc_edge_layer(n_nodes: int, e_pad: int, d: int):
    nw = NC * NS
    n_chunks = e_pad // CHUNK
    ch_per_w = n_chunks // nw
    n_grp = ch_per_w // GRP
    assert ch_per_w % (2 * GRP) == 0
    stripe = (n_nodes // (8 * NS)) * 8
    last_stripe = n_nodes - stripe * (NS - 1)
    d_vecs = d // LANES

    mesh = plsc.VectorSubcoreMesh(
        core_axis_name="c", subcore_axis_name="s", num_cores=NC, num_subcores=NS
    )

    @functools.partial(
        pl.kernel,
        out_type=jax.ShapeDtypeStruct((NC, n_nodes, d), jnp.float32),
        mesh=mesh,
        scratch_types=[
            pltpu.VMEM_SHARED((n_nodes, d), jnp.float32),
            pltpu.VMEM((GRP, 2, CHUNK), jnp.int32),
            pltpu.VMEM((GRP, 2, CHUNK), jnp.int32),
            pltpu.VMEM((GRP, CHUNK), jnp.float32),
            pltpu.VMEM((GRP, CHUNK), jnp.float32),
            pltpu.VMEM((CHUNK, d), jnp.float32),
            pltpu.VMEM((CHUNK, d), jnp.float32),
            pltpu.SemaphoreType.DMA,
            pltpu.SemaphoreType.DMA,
            pltpu.SemaphoreType.DMA,
            pltpu.SemaphoreType.DMA,
        ],
    )
    def sc_layer(h_hbm, edges_hbm, ew_hbm, zeros_hbm, out_hbm,
                 acc, ig0, ig1, ie0, ie1, rows_a, rows_b, si0, si1, sa, sb):
        cid = lax.axis_index("c")
        sid = lax.axis_index("s")
        wid = cid * NS + sid
        row_base = sid * stripe
        grp_base = wid * n_grp

        def idx_hbm(g_abs):
            return edges_hbm.at[pl.ds(g_abs * GRP, GRP)]

        def ewg_hbm(g_abs):
            return ew_hbm.at[pl.ds(g_abs * GRP, GRP)]

        def fire_idx(g_abs, buf, ebuf, sem):
            pltpu.async_copy(idx_hbm(g_abs), buf, sem)
            pltpu.async_copy(ewg_hbm(g_abs), ebuf, sem)

        def wait_idx(g_abs, buf, ebuf, sem):
            pltpu.make_async_copy(idx_hbm(g_abs), buf, sem).wait()
            pltpu.make_async_copy(ewg_hbm(g_abs), ebuf, sem).wait()

        def fire_g(buf_idx, j, rows, sem):
            pltpu.async_copy(h_hbm.at[buf_idx.at[j, 0]], rows, sem)

        def wait_g(buf_idx, j, rows, sem):
            pltpu.make_async_copy(h_hbm.at[buf_idx.at[j, 0]], rows, sem).wait()

        def consume(buf_idx, buf_ew, j, rows, sem):
            wait_g(buf_idx, j, rows, sem)

            @pl.loop(0, CHUNK // LANES)
            def _scale(g):
                w16 = buf_ew[j, pl.ds(g * LANES, LANES)]
                for k in range(LANES):
                    w = w16[k]
                    e = g * LANES + k
                    for jj in range(d_vecs):
                        sl = pl.ds(jj * LANES, LANES)
                        rows[e, sl] = rows[e, sl] * w

            pltpu.sync_copy(rows, acc.at[buf_idx.at[j, 1]], add=True)

        fire_idx(grp_base, ig0, ie0, si0)
        fire_idx(grp_base + 1, ig1, ie1, si1)
        wait_idx(grp_base, ig0, ie0, si0)
        fire_g(ig0, 0, rows_a, sa)

        @pl.when(sid < NS - 1)
        def _zero_acc():
            pltpu.sync_copy(zeros_hbm.at[pl.ds(0, stripe)],
                            acc.at[pl.ds(row_base, stripe)])

        @pl.when(sid == NS - 1)
        def _zero_acc_last():
            pltpu.sync_copy(zeros_hbm.at[pl.ds(0, last_stripe)],
                            acc.at[pl.ds(row_base, last_stripe)])

        plsc.subcore_barrier()

        rows_sem = [(rows_a, sa), (rows_b, sb)]

        @pl.loop(0, n_grp // 2)
        def _pair(k):
            g0 = grp_base + 2 * k
            not_last = k < (n_grp // 2 - 1)

            def half(g_abs, buf, ebuf, buf_nxt, ebuf_nxt, sem_nxt, nxt_exists):
                for j in range(GRP):
                    cur_rows, cur_sem = rows_sem[j % 2]
                    nxt_rows, nxt_sem = rows_sem[(j + 1) % 2]
                    if j < GRP - 1:
                        fire_g(buf, j + 1, nxt_rows, nxt_sem)
                    else:
                        @pl.when(nxt_exists)
                        def _next_grp_gather():
                            wait_idx(g_abs + 1, buf_nxt, ebuf_nxt, sem_nxt)
                            fire_g(buf_nxt, 0, nxt_rows, nxt_sem)
                    consume(buf, ebuf, j, cur_rows, cur_sem)

            half(g0, ig0, ie0, ig1, ie1, si1, True)

            @pl.when(not_last)
            def _prefetch_even():
                fire_idx(g0 + 2, ig0, ie0, si0)

            half(g0 + 1, ig1, ie1, ig0, ie0, si0, not_last)

            @pl.when(not_last)
            def _prefetch_odd():
                fire_idx(g0 + 3, ig1, ie1, si1)

        plsc.subcore_barrier()

        @pl.when(sid < NS - 1)
        def _writeout():
            pltpu.sync_copy(acc.at[pl.ds(row_base, stripe)],
                            out_hbm.at[cid].at[pl.ds(row_base, stripe)])

        @pl.when(sid == NS - 1)
        def _writeout_last():
            pltpu.sync_copy(acc.at[pl.ds(row_base, last_stripe)],
                            out_hbm.at[cid].at[pl.ds(row_base, last_stripe)])

    return sc_layer


@functools.lru_cache(maxsize=None)
def _make_tc_dense_layer(n_nodes: int, d: int):
    blk = 2000
    assert n_nodes % blk == 0
    grid = n_nodes // blk

    def body(p_ref, wc_ref, bc_ref, wm_ref, bm_ref, o_ref):
        agg = p_ref[0] + p_ref[1]
        t = jnp.dot(agg, wc_ref[...], preferred_element_type=jnp.float32)
        t = t + bc_ref[...]
        y = jnp.dot(t, wm_ref[...], preferred_element_type=jnp.float32)
        y = y + bm_ref[...]
        o_ref[...] = jnp.where(y > 0, y, 0.01 * y)

    return pl.pallas_call(
        body,
        grid=(grid,),
        in_specs=[
            pl.BlockSpec((NC, blk, d), lambda i: (0, i, 0)),
            pl.BlockSpec((d, d), lambda i: (0, 0)),
            pl.BlockSpec((1, d), lambda i: (0, 0)),
            pl.BlockSpec((d, d), lambda i: (0, 0)),
            pl.BlockSpec((1, d), lambda i: (0, 0)),
        ],
        out_specs=pl.BlockSpec((blk, d), lambda i: (i, 0)),
        out_shape=jax.ShapeDtypeStruct((n_nodes, d), jnp.float32),
    )


def kernel(x, edge_index, edge_weight, Wc, bc, Wm, bm):
    n, d = x.shape
    e = edge_weight.shape[0]
    e_pad = _ceil_to(e, 2 * GRP * NC * NS * CHUNK)
    pad = e_pad - e
    pad_idx = (jnp.arange(pad, dtype=jnp.int32) % n) if pad else jnp.zeros((0,), jnp.int32)
    src = jnp.concatenate([edge_index[0].astype(jnp.int32), pad_idx]).reshape(-1, CHUNK)
    dst = jnp.concatenate([edge_index[1].astype(jnp.int32), pad_idx]).reshape(-1, CHUNK)
    ew2d = jnp.pad(edge_weight.astype(jnp.float32), (0, pad)).reshape(-1, CHUNK)
    edges = jnp.stack([src, dst], axis=1)

    sc_layer = _make_sc_edge_layer(n, e_pad, d)
    tc_layer = _make_tc_dense_layer(n, d)
    nz = n - (n // (8 * NS)) * 8 * (NS - 1)
    zeros = jnp.zeros((nz, d), jnp.float32)

    h = x
    for l in range(Wc.shape[0]):
        parts = sc_layer(h, edges, ew2d, zeros)
        h = tc_layer(parts, Wc[l], bc[l].reshape(1, d), Wm[l], bm[l].reshape(1, d))
    return h


# speedup vs baseline: 1.0178x; 1.0178x over previous
import functools

import jax
import jax.numpy as jnp
from jax import lax
from jax.experimental import pallas as pl
from jax.experimental.pallas import tpu as pltpu
from jax.experimental.pallas import tpu_sc as plsc

NC = 2
NS = 16
LANES = 16
CHUNK = 128
GRP = 4


def _ceil_to(x, m):
    return (x + m - 1) // m * m


@functools.lru_cache(maxsize=None)
def _make_sc_edge_layer(n_nodes: int, e_pad: int, d: int):
    nw = NC * NS
    n_chunks = e_pad // CHUNK
    ch_per_w = n_chunks // nw
    n_grp = ch_per_w // GRP
    assert ch_per_w % (2 * GRP) == 0
    stripe = (n_nodes // (8 * NS)) * 8
    last_stripe = n_nodes - stripe * (NS - 1)
    d_vecs = d // LANES

    mesh = plsc.VectorSubcoreMesh(
        core_axis_name="c", subcore_axis_name="s", num_cores=NC, num_subcores=NS
    )

    @functools.partial(
        pl.kernel,
        out_type=jax.ShapeDtypeStruct((NC, n_nodes, d), jnp.float32),
        mesh=mesh,
        scratch_types=[
            pltpu.VMEM_SHARED((n_nodes, d), jnp.float32),
            pltpu.VMEM((GRP, 2, CHUNK), jnp.int32),
            pltpu.VMEM((GRP, 2, CHUNK), jnp.int32),
            pltpu.VMEM((GRP, CHUNK), jnp.float32),
            pltpu.VMEM((GRP, CHUNK), jnp.float32),
            pltpu.VMEM((CHUNK, d), jnp.float32),
            pltpu.VMEM((CHUNK, d), jnp.float32),
            pltpu.SemaphoreType.DMA,
            pltpu.SemaphoreType.DMA,
            pltpu.SemaphoreType.DMA,
            pltpu.SemaphoreType.DMA,
        ],
    )
    def sc_layer(h_hbm, edges_hbm, ew_hbm, zeros_hbm, out_hbm,
                 acc, ig0, ig1, ie0, ie1, rows_a, rows_b, si0, si1, sa, sb):
        cid = lax.axis_index("c")
        sid = lax.axis_index("s")
        wid = cid * NS + sid
        row_base = sid * stripe
        grp_base = wid * n_grp

        def idx_hbm(g_abs):
            return edges_hbm.at[pl.ds(g_abs * GRP, GRP)]

        def ewg_hbm(g_abs):
            return ew_hbm.at[pl.ds(g_abs * GRP, GRP)]

        def fire_idx(g_abs, buf, ebuf, sem):
            pltpu.async_copy(idx_hbm(g_abs), buf, sem)
            pltpu.async_copy(ewg_hbm(g_abs), ebuf, sem)

        def wait_idx(g_abs, buf, ebuf, sem):
            pltpu.make_async_copy(idx_hbm(g_abs), buf, sem).wait()
            pltpu.make_async_copy(ewg_hbm(g_abs), ebuf, sem).wait()

        def fire_g(buf_idx, j, rows, sem):
            pltpu.async_copy(h_hbm.at[buf_idx.at[j, 0]], rows, sem)

        def wait_g(buf_idx, j, rows, sem):
            pltpu.make_async_copy(h_hbm.at[buf_idx.at[j, 0]], rows, sem).wait()

        def consume(buf_idx, buf_ew, j, rows, sem):
            wait_g(buf_idx, j, rows, sem)

            @pl.loop(0, CHUNK // LANES)
            def _scale(g):
                w16 = buf_ew[j, pl.ds(g * LANES, LANES)]
                for k in range(LANES):
                    w = w16[k]
                    e = g * LANES + k
                    for jj in range(d_vecs):
                        sl = pl.ds(jj * LANES, LANES)
                        rows[e, sl] = rows[e, sl] * w

            pltpu.sync_copy(rows, acc.at[buf_idx.at[j, 1]], add=True)

        fire_idx(grp_base, ig0, ie0, si0)
        fire_idx(grp_base + 1, ig1, ie1, si1)
        wait_idx(grp_base, ig0, ie0, si0)
        fire_g(ig0, 0, rows_a, sa)

        @pl.when(sid < NS - 1)
        def _zero_acc():
            pltpu.sync_copy(zeros_hbm.at[pl.ds(0, stripe)],
                            acc.at[pl.ds(row_base, stripe)])

        @pl.when(sid == NS - 1)
        def _zero_acc_last():
            pltpu.sync_copy(zeros_hbm.at[pl.ds(0, last_stripe)],
                            acc.at[pl.ds(row_base, last_stripe)])

        plsc.subcore_barrier()

        rows_sem = [(rows_a, sa), (rows_b, sb)]

        @pl.loop(0, n_grp // 2)
        def _pair(k):
            g0 = grp_base + 2 * k
            not_last = k < (n_grp // 2 - 1)

            def half(g_abs, buf, ebuf, buf_nxt, ebuf_nxt, sem_nxt, nxt_exists):
                for j in range(GRP):
                    cur_rows, cur_sem = rows_sem[j % 2]
                    nxt_rows, nxt_sem = rows_sem[(j + 1) % 2]
                    if j < GRP - 1:
                        fire_g(buf, j + 1, nxt_rows, nxt_sem)
                    else:
                        @pl.when(nxt_exists)
                        def _next_grp_gather():
                            wait_idx(g_abs + 1, buf_nxt, ebuf_nxt, sem_nxt)
                            fire_g(buf_nxt, 0, nxt_rows, nxt_sem)
                    consume(buf, ebuf, j, cur_rows, cur_sem)

            half(g0, ig0, ie0, ig1, ie1, si1, True)

            @pl.when(not_last)
            def _prefetch_even():
                fire_idx(g0 + 2, ig0, ie0, si0)

            half(g0 + 1, ig1, ie1, ig0, ie0, si0, not_last)

            @pl.when(not_last)
            def _prefetch_odd():
                fire_idx(g0 + 3, ig1, ie1, si1)

        plsc.subcore_barrier()

        @pl.when(sid < NS - 1)
        def _writeout():
            pltpu.sync_copy(acc.at[pl.ds(row_base, stripe)],
                            out_hbm.at[cid].at[pl.ds(row_base, stripe)])

        @pl.when(sid == NS - 1)
        def _writeout_last():
            pltpu.sync_copy(acc.at[pl.ds(row_base, last_stripe)],
                            out_hbm.at[cid].at[pl.ds(row_base, last_stripe)])

    return sc_layer


@functools.lru_cache(maxsize=None)
def _make_tc_dense_layer(n_nodes: int, d: int):
    blk = 2000
    assert n_nodes % blk == 0
    grid = n_nodes // blk

    def body(p_ref, wc_ref, bc_ref, wm_ref, bm_ref, o_ref):
        agg = p_ref[0] + p_ref[1]
        t = jnp.dot(agg, wc_ref[...], preferred_element_type=jnp.float32)
        t = t + bc_ref[...]
        y = jnp.dot(t, wm_ref[...], preferred_element_type=jnp.float32)
        y = y + bm_ref[...]
        o_ref[...] = jnp.where(y > 0, y, 0.01 * y)

    return pl.pallas_call(
        body,
        grid=(grid,),
        in_specs=[
            pl.BlockSpec((NC, blk, d), lambda i: (0, i, 0)),
            pl.BlockSpec((d, d), lambda i: (0, 0)),
            pl.BlockSpec((1, d), lambda i: (0, 0)),
            pl.BlockSpec((d, d), lambda i: (0, 0)),
            pl.BlockSpec((1, d), lambda i: (0, 0)),
        ],
        out_specs=pl.BlockSpec((blk, d), lambda i: (i, 0)),
        out_shape=jax.ShapeDtypeStruct((n_nodes, d), jnp.float32),
    )


def kernel(x, edge_index, edge_weight, Wc, bc, Wm, bm):
    n, d = x.shape
    e = edge_weight.shape[0]
    e_pad = _ceil_to(e, 2 * GRP * NC * NS * CHUNK)
    pad = e_pad - e
    pad_idx = (jnp.arange(pad, dtype=jnp.int32) % n) if pad else jnp.zeros((0,), jnp.int32)
    src = jnp.concatenate([edge_index[0].astype(jnp.int32), pad_idx]).reshape(-1, CHUNK)
    dst = jnp.concatenate([edge_index[1].astype(jnp.int32), pad_idx]).reshape(-1, CHUNK)
    ew2d = jnp.pad(edge_weight.astype(jnp.float32), (0, pad)).reshape(-1, CHUNK)
    edges = jnp.stack([src, dst], axis=1)

    sc_layer = _make_sc_edge_layer(n, e_pad, d)
    tc_layer = _make_tc_dense_layer(n, d)
    nz = n - (n // (8 * NS)) * 8 * (NS - 1)
    zeros = jnp.zeros((nz, d), jnp.float32)

    h = x
    for l in range(Wc.shape[0]):
        parts = sc_layer(h, edges, ew2d, zeros)
        h = tc_layer(parts, Wc[l], bc[l].reshape(1, d), Wm[l], bm[l].reshape(1, d))
    return h
